# Initial kernel scaffold; baseline (speedup 1.0000x reference)
import jax
import jax.numpy as jnp
from jax.experimental import pallas as pl


def kernel(region_sequences, time_sequences, region_table, time_table, W_ih, W_hh, b_ih, b_hh, fc_W, fc_b):
    raise NotImplementedError("write your pallas kernel here")


# speedup vs baseline: 3.2123x; 3.2123x over previous
import functools

import jax
import jax.numpy as jnp
from jax import lax
from jax.experimental import pallas as pl
from jax.experimental.pallas import tpu as pltpu
from jax.experimental.pallas import tpu_sc as plsc

B = 1024
L = 50
RDIM = 64
TDIM = 64
D = RDIM + TDIM
H = 256
OUT = 128
LB = L * B

_NC = 2
_NS = 16
_NW = _NC * _NS
_BPW = LB // _NW
_CH = 80
_NCH = _BPW // _CH
_HALF = _NCH // 2
_ROWS = _HALF * _CH


def _sc_gather(idx3d, ctab):
    mesh = plsc.VectorSubcoreMesh(core_axis_name="c", subcore_axis_name="s")

    @functools.partial(
        pl.kernel,
        out_type=jax.ShapeDtypeStruct((LB, D), jnp.float32),
        mesh=mesh,
        scratch_types=(
            pltpu.VMEM((_NCH, _CH), jnp.int32),
            pltpu.VMEM((_ROWS, D), jnp.float32),
            pltpu.SemaphoreType.DMA,
        ),
    )
    def k(idx_hbm, tab_hbm, out_hbm, idx_v, rows_v, sem):
        wid = lax.axis_index("s") * _NC + lax.axis_index("c")
        base = wid * _BPW
        pltpu.sync_copy(idx_hbm.at[wid], idx_v)
        for half in range(2):
            descs = [
                pltpu.async_copy(
                    tab_hbm.at[idx_v.at[half * _HALF + j]],
                    rows_v.at[pl.ds(j * _CH, _CH)],
                    sem,
                )
                for j in range(_HALF)
            ]
            for d in descs:
                d.wait()
            pltpu.sync_copy(
                rows_v, out_hbm.at[pl.ds(base + half * _ROWS, _ROWS)]
            )

    return k(idx3d, ctab)


def _lstm_body(x_ref, tv_ref, wih_ref, whh_ref, b_ref, fcw_ref, fcb_ref,
               out_ref, h_scr, c_scr):
    t = pl.program_id(0)

    @pl.when(t == 0)
    def _():
        h_scr[...] = jnp.zeros_like(h_scr)
        c_scr[...] = jnp.zeros_like(c_scr)

    x = x_ref[0]
    tv = tv_ref[0]
    col = lax.broadcasted_iota(jnp.int32, (B, D), 1)
    xs = jnp.where(col < RDIM, x, x * tv)
    h = h_scr[...]
    gates = (
        jnp.dot(xs, wih_ref[...], preferred_element_type=jnp.float32)
        + jnp.dot(h, whh_ref[...], preferred_element_type=jnp.float32)
        + b_ref[...]
    )
    i = jax.nn.sigmoid(gates[:, :H])
    f = jax.nn.sigmoid(gates[:, H:2 * H])
    g = jnp.tanh(gates[:, 2 * H:3 * H])
    o = jax.nn.sigmoid(gates[:, 3 * H:])
    c = f * c_scr[...] + i * g
    hn = o * jnp.tanh(c)
    c_scr[...] = c
    h_scr[...] = hn

    @pl.when(t == L - 1)
    def _():
        out_ref[...] = jax.nn.sigmoid(
            jnp.dot(hn, fcw_ref[...], preferred_element_type=jnp.float32)
            + fcb_ref[...]
        )


def _lstm_tc(x, tv, wih_t, whh_t, bias, fcw_t, fcb, interpret=False):
    return pl.pallas_call(
        _lstm_body,
        grid=(L,),
        in_specs=[
            pl.BlockSpec((1, B, D), lambda t: (t, 0, 0)),
            pl.BlockSpec((1, B, 1), lambda t: (t, 0, 0)),
            pl.BlockSpec((D, 4 * H), lambda t: (0, 0)),
            pl.BlockSpec((H, 4 * H), lambda t: (0, 0)),
            pl.BlockSpec((1, 4 * H), lambda t: (0, 0)),
            pl.BlockSpec((H, OUT), lambda t: (0, 0)),
            pl.BlockSpec((1, OUT), lambda t: (0, 0)),
        ],
        out_specs=pl.BlockSpec((B, OUT), lambda t: (0, 0)),
        out_shape=jax.ShapeDtypeStruct((B, OUT), jnp.float32),
        scratch_shapes=[
            pltpu.VMEM((B, H), jnp.float32),
            pltpu.VMEM((B, H), jnp.float32),
        ],
        interpret=interpret,
    )(x, tv, wih_t, whh_t, bias, fcw_t, fcb)


def kernel(region_sequences, time_sequences, region_table, time_table,
           W_ih, W_hh, b_ih, b_hh, fc_W, fc_b):
    idx = jnp.transpose(region_sequences).reshape(_NW, _NCH, _CH)
    ctab = jnp.concatenate([region_table, time_table], axis=1)
    x = _sc_gather(idx, ctab).reshape(L, B, D)
    tv = jnp.transpose(time_sequences).reshape(L, B, 1)
    wih_t = W_ih.T
    whh_t = W_hh.T
    bias = (b_ih + b_hh).reshape(1, 4 * H)
    fcw_t = fc_W.T
    fcb = fc_b.reshape(1, OUT)
    return _lstm_tc(x, tv, wih_t, whh_t, bias, fcw_t, fcb)


# speedup vs baseline: 3.2159x; 1.0011x over previous
import functools

import jax
import jax.numpy as jnp
from jax import lax
from jax.experimental import pallas as pl
from jax.experimental.pallas import tpu as pltpu
from jax.experimental.pallas import tpu_sc as plsc

B = 1024
L = 50
RDIM = 64
TDIM = 64
D = RDIM + TDIM
H = 256
OUT = 128
LB = L * B

_NC = 2
_NS = 16
_NW = _NC * _NS
_BPW = LB // _NW
_CH = 80
_NCH = _BPW // _CH
_HALF = _NCH // 2
_ROWS = _HALF * _CH


def _sc_gather(idx3d, ctab):
    mesh = plsc.VectorSubcoreMesh(core_axis_name="c", subcore_axis_name="s")

    @functools.partial(
        pl.kernel,
        out_type=jax.ShapeDtypeStruct((LB, D), jnp.float32),
        mesh=mesh,
        scratch_types=(
            pltpu.VMEM((_NCH, _CH), jnp.int32),
            pltpu.VMEM((_ROWS, D), jnp.float32),
            pltpu.SemaphoreType.DMA,
        ),
    )
    def k(idx_hbm, tab_hbm, out_hbm, idx_v, rows_v, sem):
        wid = lax.axis_index("s") * _NC + lax.axis_index("c")
        base = wid * _BPW
        pltpu.sync_copy(idx_hbm.at[wid], idx_v)
        for half in range(2):
            descs = [
                pltpu.async_copy(
                    tab_hbm.at[idx_v.at[half * _HALF + j]],
                    rows_v.at[pl.ds(j * _CH, _CH)],
                    sem,
                )
                for j in range(_HALF)
            ]
            for d in descs:
                d.wait()
            pltpu.sync_copy(
                rows_v, out_hbm.at[pl.ds(base + half * _ROWS, _ROWS)]
            )

    return k(idx3d, ctab)


def _lstm_body(x_ref, tv_ref, wih_ref, whh_ref, b_ref, fcw_ref, fcb_ref,
               out_ref, h_scr, c_scr):
    t = pl.program_id(0)

    @pl.when(t == 0)
    def _():
        h_scr[...] = jnp.zeros_like(h_scr)
        c_scr[...] = jnp.zeros_like(c_scr)

    x = x_ref[0]
    tv = tv_ref[0]
    col = lax.broadcasted_iota(jnp.int32, (B, D), 1)
    xs = jnp.where(col < RDIM, x, x * tv).astype(jnp.bfloat16)
    h = h_scr[...]
    gates = (
        jnp.dot(xs, wih_ref[...], preferred_element_type=jnp.float32)
        + jnp.dot(h.astype(jnp.bfloat16), whh_ref[...],
                  preferred_element_type=jnp.float32)
        + b_ref[...]
    )
    i = jax.nn.sigmoid(gates[:, :H])
    f = jax.nn.sigmoid(gates[:, H:2 * H])
    g = jnp.tanh(gates[:, 2 * H:3 * H])
    o = jax.nn.sigmoid(gates[:, 3 * H:])
    c = f * c_scr[...] + i * g
    hn = o * jnp.tanh(c)
    c_scr[...] = c
    h_scr[...] = hn

    @pl.when(t == L - 1)
    def _():
        out_ref[...] = jax.nn.sigmoid(
            jnp.dot(hn.astype(jnp.bfloat16), fcw_ref[...],
                    preferred_element_type=jnp.float32)
            + fcb_ref[...]
        )


def _lstm_tc(x, tv, wih_t, whh_t, bias, fcw_t, fcb, interpret=False):
    return pl.pallas_call(
        _lstm_body,
        grid=(L,),
        in_specs=[
            pl.BlockSpec((1, B, D), lambda t: (t, 0, 0)),
            pl.BlockSpec((1, B, 1), lambda t: (t, 0, 0)),
            pl.BlockSpec((D, 4 * H), lambda t: (0, 0)),
            pl.BlockSpec((H, 4 * H), lambda t: (0, 0)),
            pl.BlockSpec((1, 4 * H), lambda t: (0, 0)),
            pl.BlockSpec((H, OUT), lambda t: (0, 0)),
            pl.BlockSpec((1, OUT), lambda t: (0, 0)),
        ],
        out_specs=pl.BlockSpec((B, OUT), lambda t: (0, 0)),
        out_shape=jax.ShapeDtypeStruct((B, OUT), jnp.float32),
        scratch_shapes=[
            pltpu.VMEM((B, H), jnp.float32),
            pltpu.VMEM((B, H), jnp.float32),
        ],
        interpret=interpret,
    )(x, tv, wih_t, whh_t, bias, fcw_t, fcb)


def kernel(region_sequences, time_sequences, region_table, time_table,
           W_ih, W_hh, b_ih, b_hh, fc_W, fc_b):
    idx = jnp.transpose(region_sequences).reshape(_NW, _NCH, _CH)
    ctab = jnp.concatenate([region_table, time_table], axis=1)
    x = _sc_gather(idx, ctab).reshape(L, B, D)
    tv = jnp.transpose(time_sequences).reshape(L, B, 1)
    wih_t = W_ih.T.astype(jnp.bfloat16)
    whh_t = W_hh.T.astype(jnp.bfloat16)
    bias = (b_ih + b_hh).reshape(1, 4 * H)
    fcw_t = fc_W.T.astype(jnp.bfloat16)
    fcb = fc_b.reshape(1, OUT)
    return _lstm_tc(x, tv, wih_t, whh_t, bias, fcw_t, fcb)


# speedup vs baseline: 3.4351x; 1.0682x over previous
import functools

import jax
import jax.numpy as jnp
from jax import lax
from jax.experimental import pallas as pl
from jax.experimental.pallas import tpu as pltpu
from jax.experimental.pallas import tpu_sc as plsc

B = 1024
L = 50
RDIM = 64
TDIM = 64
D = RDIM + TDIM
H = 256
OUT = 128
LB = L * B

_NC = 2
_NS = 16
_NW = _NC * _NS
_BPW = LB // _NW
_CH = 80
_NCH = _BPW // _CH
_HALF = _NCH // 2
_ROWS = _HALF * _CH


def _sc_gather(idx3d, ctab):
    mesh = plsc.VectorSubcoreMesh(core_axis_name="c", subcore_axis_name="s")

    @functools.partial(
        pl.kernel,
        out_type=jax.ShapeDtypeStruct((LB, D), jnp.float32),
        mesh=mesh,
        scratch_types=(
            pltpu.VMEM((_NCH, _CH), jnp.int32),
            pltpu.VMEM((_ROWS, D), jnp.float32),
            pltpu.SemaphoreType.DMA,
        ),
    )
    def k(idx_hbm, tab_hbm, out_hbm, idx_v, rows_v, sem):
        wid = lax.axis_index("s") * _NC + lax.axis_index("c")
        base = wid * _BPW
        pltpu.sync_copy(idx_hbm.at[wid], idx_v)
        for half in range(2):
            descs = [
                pltpu.async_copy(
                    tab_hbm.at[idx_v.at[half * _HALF + j]],
                    rows_v.at[pl.ds(j * _CH, _CH)],
                    sem,
                )
                for j in range(_HALF)
            ]
            for d in descs:
                d.wait()
            pltpu.sync_copy(
                rows_v, out_hbm.at[pl.ds(base + half * _ROWS, _ROWS)]
            )

    return k(idx3d, ctab)


def _lstm_body(x_ref, tv_ref, wih_ref, whh_ref, b_ref, fcw_ref, fcb_ref,
               out_ref, h_scr, c_scr):
    t = pl.program_id(0)

    @pl.when(t == 0)
    def _():
        h_scr[...] = jnp.zeros_like(h_scr)
        c_scr[...] = jnp.zeros_like(c_scr)

    x = x_ref[0]
    tm = tv_ref[0]
    xs = jnp.concatenate(
        [x[:, :RDIM], x[:, RDIM:] * tm], axis=1
    ).astype(jnp.bfloat16)
    h = h_scr[...]
    gates = (
        jnp.dot(xs, wih_ref[...], preferred_element_type=jnp.float32)
        + jnp.dot(h.astype(jnp.bfloat16), whh_ref[...],
                  preferred_element_type=jnp.float32)
        + b_ref[...]
    )
    th = jnp.tanh(gates)
    i = th[:, :H] * 0.5 + 0.5
    f = th[:, H:2 * H] * 0.5 + 0.5
    g = th[:, 2 * H:3 * H]
    o = th[:, 3 * H:] * 0.5 + 0.5
    c = f * c_scr[...] + i * g
    hn = o * jnp.tanh(c)
    c_scr[...] = c
    h_scr[...] = hn

    @pl.when(t == L - 1)
    def _():
        out_ref[...] = jnp.tanh(
            jnp.dot(hn.astype(jnp.bfloat16), fcw_ref[...],
                    preferred_element_type=jnp.float32)
            + fcb_ref[...]
        ) * 0.5 + 0.5


def _lstm_tc(x, tv, wih_t, whh_t, bias, fcw_t, fcb, interpret=False):
    return pl.pallas_call(
        _lstm_body,
        grid=(L,),
        in_specs=[
            pl.BlockSpec((1, B, D), lambda t: (t, 0, 0)),
            pl.BlockSpec((1, B, TDIM), lambda t: (t, 0, 0)),
            pl.BlockSpec((D, 4 * H), lambda t: (0, 0)),
            pl.BlockSpec((H, 4 * H), lambda t: (0, 0)),
            pl.BlockSpec((1, 4 * H), lambda t: (0, 0)),
            pl.BlockSpec((H, OUT), lambda t: (0, 0)),
            pl.BlockSpec((1, OUT), lambda t: (0, 0)),
        ],
        out_specs=pl.BlockSpec((B, OUT), lambda t: (0, 0)),
        out_shape=jax.ShapeDtypeStruct((B, OUT), jnp.float32),
        scratch_shapes=[
            pltpu.VMEM((B, H), jnp.float32),
            pltpu.VMEM((B, H), jnp.float32),
        ],
        interpret=interpret,
    )(x, tv, wih_t, whh_t, bias, fcw_t, fcb)


def kernel(region_sequences, time_sequences, region_table, time_table,
           W_ih, W_hh, b_ih, b_hh, fc_W, fc_b):
    idx = jnp.transpose(region_sequences).reshape(_NW, _NCH, _CH)
    ctab = jnp.concatenate([region_table, time_table], axis=1)
    x = _sc_gather(idx, ctab).reshape(L, B, D)
    tv = jnp.broadcast_to(
        jnp.transpose(time_sequences).reshape(L, B, 1), (L, B, TDIM)
    )
    colscale = jnp.concatenate(
        [jnp.full((2 * H,), 0.5, jnp.float32),
         jnp.ones((H,), jnp.float32),
         jnp.full((H,), 0.5, jnp.float32)]
    )
    wih_t = (W_ih.T * colscale[None, :]).astype(jnp.bfloat16)
    whh_t = (W_hh.T * colscale[None, :]).astype(jnp.bfloat16)
    bias = ((b_ih + b_hh) * colscale).reshape(1, 4 * H)
    fcw_t = (fc_W.T * 0.5).astype(jnp.bfloat16)
    fcb = (fc_b * 0.5).reshape(1, OUT)
    return _lstm_tc(x, tv, wih_t, whh_t, bias, fcw_t, fcb)
